# baseline (speedup 1.0000x reference)
import jax
import jax.numpy as jnp
from jax import lax
from jax.experimental import pallas as pl
from jax.experimental.pallas import tpu as pltpu
from jax.experimental.pallas import tpu_sc as plsc

_DIM = 32
_EPS = 1e-12
_NE = 1000000
_B = 16384
_WSZ = 1024
_LASTW = 976
_LWBASE = 976 * 1024
_TAIL0 = 999936
_CAP = 12
_NSET = 4
_MAXW = 62
_NU = 25
_PAD0 = 6 * _B
_STG_ROWS = 6 * _B + 1024
_RING = 4


def _rsqrt_nr(x):
    xg = jnp.maximum(x, 1e-35)
    i = plsc.bitcast(xg, jnp.int32)
    i = 0x5F3759DF - lax.shift_right_arithmetic(i, 1)
    y = plsc.bitcast(i, jnp.float32)
    for _ in range(3):
        y = y * (1.5 - 0.5 * xg * y * y)
    return y


def _inv_clamped_norm(s):
    y = _rsqrt_nr(s)
    n = s * y
    return 1.0 / jnp.maximum(n, _EPS)


def _bcast(x):
    return jnp.full((16,), x, jnp.int32)


def _k1_body(ent_hbm, rel_hbm, refs_hbm, tails_hbm, stage_hbm,
             rb, bk0, bk1, bk2, bk3, ct0, ct1, ct2, ct3,
             wml, wslot, slab3, asm3,
             fsemA, fsemB, rsem, ssem):
    wid = lax.axis_index("s") * 2 + lax.axis_index("c")
    tbl = wid & 1
    k = lax.shift_right_logical(wid, 1)
    wlo = k * 61 + jnp.minimum(k, 1)
    nw = 61 + jnp.where(k < 1, 1, 0)
    lane = lax.iota(jnp.int32, 16)
    bks = (bk0, bk1, bk2, bk3)
    cts = (ct0, ct1, ct2, ct3)

    def zero_counts(j, _):
        for s in range(_NSET):
            cts[s][pl.ds(j * 16, 16)] = jnp.zeros((16,), jnp.int32)
        return 0
    lax.fori_loop(0, _MAXW, zero_counts, 0)

    nac = 32 - 16 * tbl

    def a_role(a):
        ra = lax.shift_right_logical(a, 3)
        ent_role = ra + jnp.where(ra >= 1, 1, 0) + jnp.where(ra >= 3, 1, 0)
        rel_role = 1 + ra * 3
        return jnp.where(tbl == 0, ent_role, rel_role)

    def fetch_chunk(a):
        pltpu.async_copy(
            refs_hbm.at[a_role(a), pl.ds((a & 7) * 2048, 2048)],
            rb.at[a & 1], rsem)

    fetch_chunk(0)

    def bin_chunk(a, _):
        par = a & 1
        pltpu.make_async_copy(refs_hbm.at[0, pl.ds(0, 2048)],
                              rb.at[0], rsem).wait()

        @pl.when(a + 1 < nac)
        def _():
            fetch_chunk(a + 1)

        slot0 = a_role(a) * _B + (a & 7) * 2048

        def bin_one(v, _):
            for s in range(_NSET):
                e = rb[par, pl.ds((v * 4 + s) * 16, 16)]
                wi = lax.shift_right_logical(e, 10)
                lwi = wi - wlo
                m = (lwi >= 0) & (lwi < nw)
                lwi = jnp.where(m, lwi, 0)
                caddr = lwi * 16 + lane
                cnt = plsc.load_gather(cts[s], [caddr], mask=m)
                cnt = jnp.where(m, jnp.minimum(cnt, _CAP - 1), 0)
                ml = e & (_WSZ - 1)
                slot = slot0 + (v * 4 + s) * 16 + lane
                word = lax.shift_left(ml, 17) | slot
                plsc.store_scatter(bks[s], [caddr * _CAP + cnt],
                                   word, mask=m)
                plsc.store_scatter(cts[s], [caddr], cnt + 1, mask=m)
            return 0

        lax.fori_loop(0, 32, bin_one, 0)
        return 0

    lax.fori_loop(0, nac, bin_chunk, 0)

    def fetch(j):
        par = j & 1
        wi = wlo + j
        base = jnp.where(wi == _LASTW, _LWBASE, wi * _WSZ)
        full = wi != _LASTW
        for p in range(2):
            sem = (fsemA, fsemB)[p]

            @pl.when((par == p) & full)
            def _(p=p, sem=sem):
                @pl.when(tbl == 0)
                def _():
                    pltpu.async_copy(ent_hbm.at[:, pl.ds(base, _WSZ)],
                                     slab3.at[p], sem)
                @pl.when(tbl == 1)
                def _():
                    pltpu.async_copy(rel_hbm.at[:, pl.ds(base, _WSZ)],
                                     slab3.at[p], sem)

            @pl.when((par == p) & jnp.logical_not(full))
            def _(p=p, sem=sem):
                @pl.when(tbl == 0)
                def _():
                    pltpu.async_copy(ent_hbm.at[:, pl.ds(_LWBASE, 512)],
                                     slab3.at[p].at[:, pl.ds(0, 512)], sem)
                @pl.when(tbl == 1)
                def _():
                    pltpu.async_copy(rel_hbm.at[:, pl.ds(_LWBASE, 512)],
                                     slab3.at[p].at[:, pl.ds(0, 512)], sem)

    fetch(0)

    def window(j, _):
        par = j & 1
        wi = wlo + j
        is_last = wi == _LASTW

        @pl.when(j + 1 < nw)
        def _():
            fetch(j + 1)

        for p in range(2):
            sem = (fsemA, fsemB)[p]

            @pl.when((par == p) & jnp.logical_not(is_last))
            def _(p=p, sem=sem):
                pltpu.make_async_copy(ent_hbm.at[:, pl.ds(0, _WSZ)],
                                      slab3.at[p], sem).wait()

            @pl.when((par == p) & is_last)
            def _(p=p, sem=sem):
                pltpu.make_async_copy(
                    ent_hbm.at[:, pl.ds(0, 512)],
                    slab3.at[p].at[:, pl.ds(0, 512)], sem).wait()
                pltpu.sync_copy(tails_hbm.at[tbl],
                                slab3.at[p].at[:, pl.ds(512, 128)])

        cb = j * 16 + lane
        cnts = [plsc.load_gather(cts[s], [cb]) for s in range(_NSET)]
        totals = cnts[0] + cnts[1] + cnts[2] + cnts[3]
        csum = plsc.cumsum(totals)
        start = csum - totals
        for s in range(_NSET):
            for u in range(_CAP):
                w = plsc.load_gather(bks[s], [cb * _CAP + u])
                m = u < cnts[s]
                pos = jnp.minimum(start + u, _NU * 16 - 17)
                plsc.store_scatter(wml, [pos],
                                   lax.shift_right_logical(w, 17), mask=m)
                plsc.store_scatter(wslot, [pos], w & 0x1FFFF, mask=m)
            start = start + cnts[s]

        wcnt = jnp.minimum(csum, _NU * 16 - 16)[15]
        wml[pl.ds(wcnt, 16)] = jnp.zeros((16,), jnp.int32)
        wslot[pl.ds(wcnt, 16)] = _PAD0 + wid * 16 + lane
        nv = lax.shift_right_logical(wcnt + 15, 4)
        lane128 = lane * 128

        def extract(u, _):
            ur = u & (_RING - 1)
            ml = wml[pl.ds(u * 16, 16)]
            slot_ref = wslot.at[pl.ds(u * 16, 16)]

            @pl.when(u >= _RING)
            def _():
                pltpu.make_async_copy(
                    asm3.at[0], stage_hbm.at[wslot.at[pl.ds(0, 16)]],
                    ssem).wait()
            urv = _bcast(0) + ur
            prv = _bcast(0) + par
            for c in range(_DIM):
                cvec = _bcast(c)
                val = plsc.load_gather(slab3, [prv, cvec, ml])
                plsc.store_scatter(asm3, [urv, lane, cvec], val)
            pltpu.async_copy(asm3.at[ur], stage_hbm.at[slot_ref], ssem)
            return 0

        lax.fori_loop(0, nv, extract, 0)

        def d(i, _):
            pltpu.make_async_copy(asm3.at[0],
                                  stage_hbm.at[wslot.at[pl.ds(0, 16)]],
                                  ssem).wait()
            return 0
        lax.fori_loop(0, jnp.minimum(nv, _RING), d, 0)
        return 0

    lax.fori_loop(0, nw, window, 0)


def _k2_body(stage_hbm, out_hbm,
             p0, p1, p2, p3, p4, p5, q0, q1, q2, q3, q4, q5,
             out_v, sem):
    wid = lax.axis_index("s") * 2 + lax.axis_index("c")
    rows2 = ((p0, p1, p2, p3, p4, p5), (q0, q1, q2, q3, q4, q5))
    lane = lax.iota(jnp.int32, 16)
    zeros = jnp.zeros((16,), jnp.float32)
    pair0 = wid * 512

    def issue(chunk, par):
        base = pair0 + chunk * 64
        for role in range(6):
            pltpu.async_copy(stage_hbm.at[pl.ds(role * _B + base, 64)],
                             rows2[par][role], sem)

    def drain(par):
        for role in range(6):
            pltpu.make_async_copy(stage_hbm.at[pl.ds(0, 64)],
                                  rows2[par][role], sem).wait()

    issue(0, 0)
    acc = zeros
    for chunk in range(8):
        par = chunk % 2
        if chunk + 1 < 8:
            issue(chunk + 1, 1 - par)
        drain(par)
        rows_v = rows2[par]

        def group(g, acc_loss, rows_v=rows_v):
            row = g * 16 + lane
            sums = [zeros] * 12
            for d in range(_DIM):
                col = _bcast(d)
                for half in range(2):
                    hv = plsc.load_gather(rows_v[3 * half + 0], [row, col])
                    rv = plsc.load_gather(rows_v[3 * half + 1], [row, col])
                    tv = plsc.load_gather(rows_v[3 * half + 2], [row, col])
                    o = 6 * half
                    sums[o + 0] = sums[o + 0] + hv * hv
                    sums[o + 1] = sums[o + 1] + rv * rv
                    sums[o + 2] = sums[o + 2] + tv * tv
                    sums[o + 3] = sums[o + 3] + hv * rv
                    sums[o + 4] = sums[o + 4] + hv * tv
                    sums[o + 5] = sums[o + 5] + rv * tv
            energies = []
            for half in range(2):
                sh, sr, st, phr, pht, prt = sums[6 * half:6 * half + 6]
                a = _inv_clamped_norm(sh)
                b = _inv_clamped_norm(sr)
                c = _inv_clamped_norm(st)
                e2 = (a * a * sh + b * b * sr + c * c * st
                      + 2.0 * (a * b * phr - a * c * pht - b * c * prt))
                e2 = jnp.maximum(e2, 0.0)
                energies.append(e2 * _rsqrt_nr(e2))
            loss = jnp.maximum(1.0 + energies[0] - energies[1], 0.0)
            return acc_loss + loss

        acc = lax.fori_loop(0, 4, group, acc)

    for h in range(8):
        out_v[pl.ds(h * 16, 16)] = acc if h == 0 else zeros
    pltpu.sync_copy(out_v, out_hbm.at[wid])


def _transe_sc(ent_t, rel_t, refs, tails3):
    mesh = plsc.VectorSubcoreMesh(core_axis_name="c", subcore_axis_name="s",
                                  num_cores=2, num_subcores=16)
    k1 = pl.kernel(
        _k1_body,
        out_type=jax.ShapeDtypeStruct((_STG_ROWS, 128), jnp.float32),
        mesh=mesh,
        scratch_types=(
            [pltpu.VMEM((2, 2048), jnp.int32)]
            + [pltpu.VMEM((_MAXW * 16 * _CAP,), jnp.int32) for _ in range(4)]
            + [pltpu.VMEM((_MAXW * 16,), jnp.int32) for _ in range(4)]
            + [pltpu.VMEM((_NU * 16,), jnp.int32) for _ in range(2)]
            + [pltpu.VMEM((2, _DIM, _WSZ), jnp.float32),
               pltpu.VMEM((_RING, 16, 128), jnp.float32)]
            + [pltpu.SemaphoreType.DMA] * 4
        ),
        compiler_params=pltpu.CompilerParams(needs_layout_passes=False),
    )
    staging = k1(ent_t, rel_t, refs, tails3)

    k2 = pl.kernel(
        _k2_body,
        out_type=jax.ShapeDtypeStruct((32, 128), jnp.float32),
        mesh=mesh,
        scratch_types=(
            [pltpu.VMEM((64, 128), jnp.float32) for _ in range(12)]
            + [pltpu.VMEM((128,), jnp.float32), pltpu.SemaphoreType.DMA]
        ),
        compiler_params=pltpu.CompilerParams(needs_layout_passes=False),
    )
    return k2(staging)


@jax.jit
def kernel(pos_triples, neg_triples, ent_emb, rel_emb):
    ent_t = ent_emb.T
    rel_t = rel_emb.T
    refs = jnp.stack([pos_triples[:, 0], pos_triples[:, 1], pos_triples[:, 2],
                      neg_triples[:, 0], neg_triples[:, 1], neg_triples[:, 2]])
    zpad = jnp.zeros((64, _DIM), jnp.float32)
    tails3 = jnp.stack(
        [jnp.concatenate([ent_emb[_TAIL0:], zpad], 0).T,
         jnp.concatenate([rel_emb[_TAIL0:], zpad], 0).T])
    partials = _transe_sc(ent_t, rel_t, refs, tails3)
    return jnp.sum(partials) * (1.0 / _B)
